# baseline (speedup 1.0000x reference)
import functools

import jax
import jax.numpy as jnp
from jax import lax
from jax.experimental import pallas as pl
from jax.experimental.pallas import tpu as pltpu
from jax.experimental.pallas import tpu_sc as plsc

_NC = 2
_NS = 16
_NW = _NC * _NS
_L = 16


def _sc_lookup(idx, table, B, V, D):
    b_per_w = B // _NW
    n_grp = b_per_w // _L
    n_chunk = 2
    g_per_chunk = n_grp // n_chunk
    w_per_chunk = g_per_chunk * _L * D

    mesh = plsc.VectorSubcoreMesh(core_axis_name="c", subcore_axis_name="s")

    @functools.partial(
        pl.kernel,
        mesh=mesh,
        compiler_params=pltpu.CompilerParams(needs_layout_passes=False),
        out_type=jax.ShapeDtypeStruct((B * D,), jnp.float32),
        scratch_types=[
            pltpu.VMEM((b_per_w,), jnp.int32),
            pltpu.VMEM((V * D,), jnp.float32),
            pltpu.VMEM((b_per_w * D,), jnp.float32),
            pltpu.SemaphoreType.DMA,
        ],
    )
    def k(idx_hbm, tab_hbm, out_hbm, idx_v, tab_v, buf, sem):
        wid = lax.axis_index("s") * _NC + lax.axis_index("c")
        base = wid * b_per_w
        pltpu.sync_copy(idx_hbm.at[pl.ds(base, b_per_w)], idx_v)
        pltpu.sync_copy(tab_hbm, tab_v)

        lane_d = lax.iota(jnp.int32, _L) * D
        grp_step = jnp.full((_L,), _L * D, jnp.int32)

        for ck in range(n_chunk):
            gvec = [
                idx_v[pl.ds((ck * g_per_chunk + g) * _L, _L)] * D
                for g in range(g_per_chunk)
            ]

            @plsc.parallel_loop(0, D, unroll=1)
            def col_loop(c, ck=ck, gvec=gvec):
                cvec = jnp.full((_L,), 0, jnp.int32) + c
                scat = lane_d + (ck * g_per_chunk * _L * D + c)
                for g in range(g_per_chunk):
                    vals = plsc.load_gather(tab_v, [gvec[g] + cvec])
                    plsc.store_scatter(buf, [scat], vals)
                    scat = scat + grp_step

            pltpu.async_copy(
                buf.at[pl.ds(ck * w_per_chunk, w_per_chunk)],
                out_hbm.at[pl.ds(base * D + ck * w_per_chunk, w_per_chunk)],
                sem,
            )

        for ck in range(n_chunk):
            pltpu.make_async_copy(
                buf.at[pl.ds(ck * w_per_chunk, w_per_chunk)],
                out_hbm.at[pl.ds(base * D + ck * w_per_chunk, w_per_chunk)],
                sem,
            ).wait()

    return k(idx, table)


def kernel(inputs, table):
    B = inputs.shape[0]
    V, D = table.shape
    idx = inputs.astype(jnp.int32).reshape(B)
    out = _sc_lookup(idx, table.reshape(V * D), B, V, D)
    return out.reshape(B, D)


# speedup vs baseline: 1.6170x; 1.6170x over previous
import functools

import jax
import jax.numpy as jnp
from jax import lax
from jax.experimental import pallas as pl
from jax.experimental.pallas import tpu as pltpu
from jax.experimental.pallas import tpu_sc as plsc

_NC = 2
_NS = 16
_NW = _NC * _NS
_L = 16


def _sc_lookup(idx, table, B, V, D):
    b_per_w = B // _NW
    n_vec = D // _L
    n_chunk = 8
    g_per_chunk = (b_per_w // _L) // n_chunk
    w_per_chunk = (b_per_w // n_chunk) * D

    mesh = plsc.VectorSubcoreMesh(core_axis_name="c", subcore_axis_name="s")

    @functools.partial(
        pl.kernel,
        mesh=mesh,
        compiler_params=pltpu.CompilerParams(needs_layout_passes=False),
        out_type=jax.ShapeDtypeStruct((B * D,), jnp.float32),
        scratch_types=[
            pltpu.VMEM((b_per_w,), jnp.int32),
            pltpu.VMEM((V * D,), jnp.float32),
            pltpu.VMEM((b_per_w * D,), jnp.float32),
            pltpu.SemaphoreType.DMA,
        ],
    )
    def k(idx_hbm, tab_hbm, out_hbm, idx_v, tab_v, buf, sem):
        wid = lax.axis_index("s") * _NC + lax.axis_index("c")
        base = wid * b_per_w
        pltpu.sync_copy(idx_hbm.at[pl.ds(base, b_per_w)], idx_v)
        pltpu.sync_copy(tab_hbm, tab_v)

        def chunk_body(ck, carry):
            @plsc.parallel_loop(0, g_per_chunk, unroll=1)
            def grp_loop(g):
                gg = ck * g_per_chunk + g
                idxv = idx_v[pl.ds(gg * _L, _L)]
                rows = [idxv[l] * D for l in range(_L)]
                for l in range(_L):
                    out_off = (gg * _L + l) * D
                    for q in range(n_vec):
                        buf[pl.ds(out_off + q * _L, _L)] = (
                            tab_v[pl.ds(rows[l] + q * _L, _L)])

            pltpu.async_copy(
                buf.at[pl.ds(ck * w_per_chunk, w_per_chunk)],
                out_hbm.at[pl.ds(base * D + ck * w_per_chunk, w_per_chunk)],
                sem,
            )
            return carry

        lax.fori_loop(0, n_chunk, chunk_body, 0)
        for ck in range(n_chunk):
            pltpu.make_async_copy(
                buf.at[pl.ds(ck * w_per_chunk, w_per_chunk)],
                out_hbm.at[pl.ds(base * D + ck * w_per_chunk, w_per_chunk)],
                sem,
            ).wait()

    return k(idx, table)


def kernel(inputs, table):
    B = inputs.shape[0]
    V, D = table.shape
    idx = inputs.astype(jnp.int32).reshape(B)
    out = _sc_lookup(idx, table.reshape(V * D), B, V, D)
    return out.reshape(B, D)
